# baseline (speedup 1.0000x reference)
import functools

import jax
import jax.numpy as jnp
from jax import lax
from jax.experimental import pallas as pl
from jax.experimental.pallas import tpu as pltpu
from jax.experimental.pallas import tpu_sc as plsc

_Q_MIN = 0.01
_N = 100000
_LANES = 128
_ROWS = 784
_NPAD = _ROWS * _LANES

_NTILES = 16
_CHUNK = _NPAD // _NTILES
_VREGS = _CHUNK // 16
_NPID = 64
_BIGIDX = 1 << 30


def _sc_argmax_kernel(beta_hbm, pid_hbm, table_hbm, rows_out, bmax_out,
                      beta_v, pid_v, bmax_v, bidx_v, stage_m_v, stage_i_v,
                      rows_v, shared_m, shared_i, sem):
    cid = lax.axis_index("c")
    sid = lax.axis_index("s")

    @pl.when(cid == 0)
    def _work():
        base = sid * _CHUNK
        pltpu.sync_copy(beta_hbm.at[pl.ds(base, _CHUNK)], beta_v)
        pltpu.sync_copy(pid_hbm.at[pl.ds(base, _CHUNK)], pid_v)

        for k in range(_NPID // 16):
            bmax_v[pl.ds(k * 16, 16)] = jnp.full((16,), -1.0, jnp.float32)
            bidx_v[pl.ds(k * 16, 16)] = jnp.full((16,), _BIGIDX, jnp.int32)

        lane = lax.broadcasted_iota(jnp.int32, (16,), 0)

        def p1_body(j, carry):
            b = beta_v[pl.ds(j * 16, 16)]
            p = pid_v[pl.ds(j * 16, 16)]

            def retry(_go):
                m = plsc.load_gather(bmax_v, [p])
                need = b > m
                plsc.store_scatter(bmax_v, [p], b, mask=need)
                m2 = plsc.load_gather(bmax_v, [p])
                return jnp.any(b > m2)

            m0 = plsc.load_gather(bmax_v, [p])
            lax.while_loop(lambda go: go, retry, jnp.any(b > m0))
            return carry

        lax.fori_loop(0, _VREGS, p1_body, jnp.int32(0))

        def p2_body(j, carry):
            b = beta_v[pl.ds(j * 16, 16)]
            p = pid_v[pl.ds(j * 16, 16)]
            idxv = base + j * 16 + lane
            m = plsc.load_gather(bmax_v, [p])
            eq = b == m

            def retry(_go):
                cur = plsc.load_gather(bidx_v, [p])
                cand = jnp.logical_and(eq, idxv < cur)
                plsc.store_scatter(bidx_v, [p], idxv, mask=cand)
                cur2 = plsc.load_gather(bidx_v, [p])
                return jnp.any(jnp.logical_and(eq, idxv < cur2))

            cur0 = plsc.load_gather(bidx_v, [p])
            lax.while_loop(lambda go: go, retry,
                           jnp.any(jnp.logical_and(eq, idxv < cur0)))
            return carry

        lax.fori_loop(0, _VREGS, p2_body, jnp.int32(0))

        pltpu.sync_copy(bmax_v, shared_m.at[sid])
        pltpu.sync_copy(bidx_v, shared_i.at[sid])

    plsc.subcore_barrier()

    @pl.when(jnp.logical_and(cid == 0, sid == 0))
    def _merge():
        pltpu.sync_copy(shared_m, stage_m_v)
        pltpu.sync_copy(shared_i, stage_i_v)
        for k in range(_NPID // 16):
            sl = pl.ds(k * 16, 16)
            fm = stage_m_v[0, sl]
            fi = stage_i_v[0, sl]
            for t in range(1, _NTILES):
                tm = stage_m_v[t, sl]
                ti = stage_i_v[t, sl]
                upd = jnp.logical_or(
                    tm > fm, jnp.logical_and(tm == fm, ti < fi))
                fm = jnp.where(upd, tm, fm)
                fi = jnp.where(upd, ti, fi)
            fi = jnp.where(fm > -0.5, fi, 0)
            bmax_v[sl] = fm
            bidx_v[sl] = fi
        pltpu.sync_copy(bmax_v, bmax_out)
        pltpu.async_copy(table_hbm.at[bidx_v], rows_v, sem).wait()
        pltpu.sync_copy(rows_v, rows_out)


_sc_argmax = functools.partial(
    pl.kernel,
    out_type=[
        jax.ShapeDtypeStruct((_NPID, 8), jnp.float32),
        jax.ShapeDtypeStruct((_NPID,), jnp.float32),
    ],
    mesh=plsc.VectorSubcoreMesh(core_axis_name="c", subcore_axis_name="s"),
    compiler_params=pltpu.CompilerParams(
        needs_layout_passes=False, use_tc_tiling_on_sc=False),
    scratch_types=[
        pltpu.VMEM((_CHUNK,), jnp.float32),
        pltpu.VMEM((_CHUNK,), jnp.int32),
        pltpu.VMEM((_NPID,), jnp.float32),
        pltpu.VMEM((_NPID,), jnp.int32),
        pltpu.VMEM((_NTILES, _NPID), jnp.float32),
        pltpu.VMEM((_NTILES, _NPID), jnp.int32),
        pltpu.VMEM((_NPID, 8), jnp.float32),
        pltpu.VMEM_SHARED((_NTILES, _NPID), jnp.float32),
        pltpu.VMEM_SHARED((_NTILES, _NPID), jnp.int32),
        pltpu.SemaphoreType.DMA,
    ],
)(_sc_argmax_kernel)


def _tc_loss_kernel(beta_ref, pid_ref, x0_ref, x1_ref, x2_ref,
                    a0_ref, a1_ref, a2_ref, ba_ref, bm_ref, out_ref):
    beta = beta_ref[...]
    pid = pid_ref[...]
    x0 = x0_ref[...]
    x1 = x1_ref[...]
    x2 = x2_ref[...]

    at = 0.5 * jnp.log((1.0 + beta) / (1.0 - beta))
    q = at * at + _Q_MIN
    ridx = lax.broadcasted_iota(jnp.int32, (_ROWS, _LANES), 0)
    cidx = lax.broadcasted_iota(jnp.int32, (_ROWS, _LANES), 1)
    valid = (ridx * _LANES + cidx) < _N
    q = jnp.where(valid, q, 0.0)

    ba = ba_ref[...]
    ata = 0.5 * jnp.log((1.0 + ba) / (1.0 - ba))
    qa_vec = jnp.where(bm_ref[...] > -0.5, ata * ata + _Q_MIN, 0.0)
    a0v = a0_ref[...]
    a1v = a1_ref[...]
    a2v = a2_ref[...]
    lane = lax.broadcasted_iota(jnp.int32, (1, _NPID), 1)

    def body(p, acc):
        onep = lane == p
        qa = jnp.sum(jnp.where(onep, qa_vec, 0.0))
        a0 = jnp.sum(jnp.where(onep, a0v, 0.0))
        a1 = jnp.sum(jnp.where(onep, a1v, 0.0))
        a2 = jnp.sum(jnp.where(onep, a2v, 0.0))
        d0 = x0 - a0
        d1 = x1 - a1
        d2c = x2 - a2
        dist2 = d0 * d0 + d1 * d1 + d2c * d2c
        norm = jnp.sqrt(dist2)
        rep = jnp.maximum(1.0 - norm, 0.0)
        val = jnp.where(pid == p, dist2, 10.0 * rep)
        return acc + qa * jnp.sum(q * val)

    total = lax.fori_loop(1, 50, body, jnp.float32(0.0))
    out_ref[0, 0] = total * (1.0 / _N)


def kernel(w, beta, x, y, particle_id):
    del w, y
    pid = particle_id.reshape(-1).astype(jnp.int32)
    pad = _NPAD - _N
    beta_flat = jnp.pad(beta, (0, pad))
    pid_flat = jnp.pad(pid, (0, pad))
    x_p = jnp.pad(x.astype(jnp.float32), ((0, pad), (0, 5)))
    table = x_p.at[:_N, 3].set(beta)

    rows, bmax = _sc_argmax(beta_flat, pid_flat, table)

    beta_p = beta_flat.reshape(_ROWS, _LANES)
    pid_p = pid_flat.reshape(_ROWS, _LANES)
    x0 = x_p[:, 0].reshape(_ROWS, _LANES)
    x1 = x_p[:, 1].reshape(_ROWS, _LANES)
    x2 = x_p[:, 2].reshape(_ROWS, _LANES)
    a0v = rows[:, 0].reshape(1, _NPID)
    a1v = rows[:, 1].reshape(1, _NPID)
    a2v = rows[:, 2].reshape(1, _NPID)
    bav = rows[:, 3].reshape(1, _NPID)
    bmv = bmax.reshape(1, _NPID)

    out = pl.pallas_call(
        _tc_loss_kernel,
        out_shape=jax.ShapeDtypeStruct((1, 1), jnp.float32),
        in_specs=[pl.BlockSpec((_ROWS, _LANES), lambda: (0, 0))] * 5
        + [pl.BlockSpec((1, _NPID), lambda: (0, 0))] * 5,
        out_specs=pl.BlockSpec(memory_space=pltpu.SMEM),
    )(beta_p, pid_p, x0, x1, x2, a0v, a1v, a2v, bav, bmv)
    return out[0, 0]


# speedup vs baseline: 7.1665x; 7.1665x over previous
import jax
import jax.numpy as jnp
from jax.experimental import pallas as pl
from jax.experimental.pallas import tpu as pltpu

_Q_MIN = 0.01
_N = 100000
_LANES = 128
_ROWS = 784
_NPAD = _ROWS * _LANES


def _loss_kernel(beta_ref, pid_ref, x0_ref, x1_ref, x2_ref, out_ref):
    beta = beta_ref[...]
    pid = pid_ref[...]
    x0 = x0_ref[...]
    x1 = x1_ref[...]
    x2 = x2_ref[...]

    at = 0.5 * jnp.log((1.0 + beta) / (1.0 - beta))
    q = at * at + _Q_MIN
    ridx = jax.lax.broadcasted_iota(jnp.int32, (_ROWS, _LANES), 0)
    cidx = jax.lax.broadcasted_iota(jnp.int32, (_ROWS, _LANES), 1)
    flat = ridx * _LANES + cidx
    q = jnp.where(flat < _N, q, 0.0)
    flat_f = flat.astype(jnp.float32)
    xx = x0 * x0 + x1 * x1 + x2 * x2
    lane = jax.lax.broadcasted_iota(jnp.int32, (1, _LANES), 1)

    def body(p, acc):
        mask = pid == p
        masked_q = jnp.where(mask, q, 0.0)
        qa = jnp.max(masked_q)
        mi = jnp.min(jnp.where(masked_q == qa, flat_f, 3.0e38))
        mi_i = mi.astype(jnp.int32)
        r = mi_i >> 7
        c = mi_i & 127
        row0 = x0_ref[pl.ds(r, 1), :]
        row1 = x1_ref[pl.ds(r, 1), :]
        row2 = x2_ref[pl.ds(r, 1), :]
        onlane = lane == c
        a0 = jnp.sum(jnp.where(onlane, row0, 0.0))
        a1 = jnp.sum(jnp.where(onlane, row1, 0.0))
        a2 = jnp.sum(jnp.where(onlane, row2, 0.0))
        aa = a0 * a0 + a1 * a1 + a2 * a2
        t = x0 * a0 + x1 * a1 + x2 * a2
        dist2 = jnp.maximum((xx - 2.0 * t) + aa, 0.0)
        norm = jnp.sqrt(dist2)
        rep10 = jnp.maximum(10.0 - 10.0 * norm, 0.0)
        val = jnp.where(mask, dist2, rep10)
        return acc + qa * jnp.sum(q * val)

    total = jax.lax.fori_loop(1, 50, body, jnp.float32(0.0))
    out_ref[0, 0] = total * (1.0 / _N)


def kernel(w, beta, x, y, particle_id):
    del w, y
    pid = particle_id.reshape(-1).astype(jnp.int32)
    pad = _NPAD - _N
    beta_p = jnp.pad(beta, (0, pad)).reshape(_ROWS, _LANES)
    pid_p = jnp.pad(pid, (0, pad)).reshape(_ROWS, _LANES)
    x_p = jnp.pad(x.astype(jnp.float32), ((0, pad), (0, 0)))
    x0 = x_p[:, 0].reshape(_ROWS, _LANES)
    x1 = x_p[:, 1].reshape(_ROWS, _LANES)
    x2 = x_p[:, 2].reshape(_ROWS, _LANES)

    out = pl.pallas_call(
        _loss_kernel,
        out_shape=jax.ShapeDtypeStruct((1, 1), jnp.float32),
        in_specs=[pl.BlockSpec((_ROWS, _LANES), lambda: (0, 0))] * 5,
        out_specs=pl.BlockSpec(memory_space=pltpu.SMEM),
    )(beta_p, pid_p, x0, x1, x2)
    return out[0, 0]


# speedup vs baseline: 7.5282x; 1.0505x over previous
import jax
import jax.numpy as jnp
from jax.experimental import pallas as pl
from jax.experimental.pallas import tpu as pltpu

_Q_MIN = 0.01
_N = 100000
_LANES = 128
_ROWS = 832
_NPAD = _ROWS * _LANES


def _tree(m, op):
    m = op(m[:416], m[416:])
    m = op(m[:208], m[208:])
    return op(m[:104], m[104:])


def _loss_kernel(beta_ref, pid_ref, x0_ref, x1_ref, x2_ref, out_ref):
    beta = beta_ref[...]
    pid = pid_ref[...]
    x0 = x0_ref[...]
    x1 = x1_ref[...]
    x2 = x2_ref[...]

    at = 0.5 * jnp.log((1.0 + beta) / (1.0 - beta))
    q = at * at + _Q_MIN
    ridx = jax.lax.broadcasted_iota(jnp.int32, (_ROWS, _LANES), 0)
    cidx = jax.lax.broadcasted_iota(jnp.int32, (_ROWS, _LANES), 1)
    flat = ridx * _LANES + cidx
    q = jnp.where(flat < _N, q, 0.0)
    flat_f = flat.astype(jnp.float32)
    xx = x0 * x0 + x1 * x1 + x2 * x2
    lane = jax.lax.broadcasted_iota(jnp.int32, (1, _LANES), 1)

    def body(p, acc8):
        mask = pid == p
        masked_q = jnp.where(mask, q, 0.0)
        qa = jnp.max(_tree(masked_q, jnp.maximum))
        mi = jnp.min(_tree(jnp.where(masked_q == qa, flat_f, 3.0e38),
                           jnp.minimum))
        mi_i = mi.astype(jnp.int32)
        r = mi_i >> 7
        c = mi_i & 127
        onlane = lane == c
        a0 = jnp.sum(jnp.where(onlane, x0_ref[pl.ds(r, 1), :], 0.0))
        a1 = jnp.sum(jnp.where(onlane, x1_ref[pl.ds(r, 1), :], 0.0))
        a2 = jnp.sum(jnp.where(onlane, x2_ref[pl.ds(r, 1), :], 0.0))
        aa = a0 * a0 + a1 * a1 + a2 * a2
        t = x0 * a0 + x1 * a1 + x2 * a2
        dist2 = jnp.maximum((xx - 2.0 * t) + aa, 0.0)
        norm = jnp.sqrt(dist2)
        rep10 = jnp.maximum(10.0 - 10.0 * norm, 0.0)
        val = jnp.where(mask, dist2, rep10)
        part = _tree(q * val, jnp.add)
        part8 = part[:8] + part[8:16] + part[16:24] + part[24:32] + \
            part[32:40] + part[40:48] + part[48:56] + part[56:64] + \
            part[64:72] + part[72:80] + part[80:88] + part[88:96] + \
            part[96:104]
        return acc8 + qa * part8

    acc8 = jax.lax.fori_loop(1, 50, body, jnp.zeros((8, _LANES), jnp.float32))
    out_ref[0, 0] = jnp.sum(acc8) * (1.0 / _N)


def kernel(w, beta, x, y, particle_id):
    del w, y
    pid = particle_id.reshape(-1).astype(jnp.int32)
    pad = _NPAD - _N
    beta_p = jnp.pad(beta, (0, pad)).reshape(_ROWS, _LANES)
    pid_p = jnp.pad(pid, (0, pad)).reshape(_ROWS, _LANES)
    x_p = jnp.pad(x.astype(jnp.float32), ((0, pad), (0, 0)))
    x0 = x_p[:, 0].reshape(_ROWS, _LANES)
    x1 = x_p[:, 1].reshape(_ROWS, _LANES)
    x2 = x_p[:, 2].reshape(_ROWS, _LANES)

    out = pl.pallas_call(
        _loss_kernel,
        out_shape=jax.ShapeDtypeStruct((1, 1), jnp.float32),
        in_specs=[pl.BlockSpec((_ROWS, _LANES), lambda: (0, 0))] * 5,
        out_specs=pl.BlockSpec(memory_space=pltpu.SMEM),
    )(beta_p, pid_p, x0, x1, x2)
    return out[0, 0]
